# Initial kernel scaffold; baseline (speedup 1.0000x reference)
import jax
import jax.numpy as jnp
from jax.experimental import pallas as pl


def kernel(text, word_mask, embedding_matrix):
    raise NotImplementedError("write your pallas kernel here")


# speedup vs baseline: 1.5782x; 1.5782x over previous
import functools

import jax
import jax.numpy as jnp
from jax import lax
from jax.experimental import pallas as pl
from jax.experimental.pallas import tpu as pltpu
from jax.experimental.pallas import tpu_sc as plsc

DIM = 64
LANES = 16
CHUNK = 128


def _sc_gather(text_flat, mask_flat, table, *, n_workers, bpw):
    nchunks = bpw // CHUNK
    n = text_flat.shape[0]
    mesh = plsc.VectorSubcoreMesh(core_axis_name="c", subcore_axis_name="s")

    @functools.partial(
        pl.kernel,
        mesh=mesh,
        compiler_params=pltpu.CompilerParams(use_tc_tiling_on_sc=False),
        out_type=jax.ShapeDtypeStruct((n, DIM), jnp.float32),
        scratch_types=[
            pltpu.VMEM((bpw,), jnp.int32),
            pltpu.VMEM((bpw,), jnp.int32),
            pltpu.VMEM((nchunks, CHUNK), jnp.int32),
            pltpu.VMEM((CHUNK, DIM), jnp.float32),
            pltpu.SemaphoreType.DMA,
        ],
    )
    def body(text_hbm, mask_hbm, table_hbm, out_hbm, text_v, mask_v, idx_v, rows_v, sem):
        nc = jax.lax.axis_size("c")
        wid = lax.axis_index("s") * nc + lax.axis_index("c")
        base = wid * bpw
        pltpu.sync_copy(text_hbm.at[pl.ds(base, bpw)], text_v)
        pltpu.sync_copy(mask_hbm.at[pl.ds(base, bpw)], mask_v)

        def compute_chunk(j, _):
            for k in range(CHUNK // LANES):
                off = j * CHUNK + k * LANES
                t = text_v[pl.ds(off, LANES)]
                m = mask_v[pl.ds(off, LANES)]
                idx_v[j, pl.ds(k * LANES, LANES)] = t * m
            return 0

        lax.fori_loop(0, nchunks, compute_chunk, 0)

        def gather_chunk(j, _):
            pltpu.async_copy(table_hbm.at[idx_v.at[j]], rows_v, sem).wait()
            pltpu.sync_copy(rows_v, out_hbm.at[pl.ds(base + j * CHUNK, CHUNK)])
            return 0

        lax.fori_loop(0, nchunks, gather_chunk, 0)

    return body


def kernel(text, word_mask, embedding_matrix):
    B, L = text.shape
    n = B * L
    n_workers = 32
    bpw = n // n_workers
    text_flat = text.reshape(n).astype(jnp.int32)
    mask_flat = word_mask.reshape(n).astype(jnp.int32)
    out = _sc_gather(text_flat, mask_flat, embedding_matrix,
                     n_workers=n_workers, bpw=bpw)(
        text_flat, mask_flat, embedding_matrix)
    return out.reshape(B, L, DIM)


# speedup vs baseline: 1.5799x; 1.0011x over previous
import functools

import jax
import jax.numpy as jnp
from jax import lax
from jax.experimental import pallas as pl
from jax.experimental.pallas import tpu as pltpu
from jax.experimental.pallas import tpu_sc as plsc

DIM = 64
LANES = 16
CHUNK = 128


NBUF = 5


def _sc_gather(text_flat, mask_flat, table, *, n_workers, bpw):
    nchunks = bpw // CHUNK
    nrounds = nchunks // NBUF
    n = text_flat.shape[0]
    mesh = plsc.VectorSubcoreMesh(core_axis_name="c", subcore_axis_name="s")

    @functools.partial(
        pl.kernel,
        mesh=mesh,
        compiler_params=pltpu.CompilerParams(use_tc_tiling_on_sc=False),
        out_type=jax.ShapeDtypeStruct((n, DIM), jnp.float32),
        scratch_types=[
            pltpu.VMEM((bpw,), jnp.int32),
            pltpu.VMEM((bpw,), jnp.int32),
            pltpu.VMEM((nchunks, CHUNK), jnp.int32),
            pltpu.VMEM((NBUF, CHUNK, DIM), jnp.float32),
            pltpu.SemaphoreType.DMA((NBUF,)),
            pltpu.SemaphoreType.DMA((NBUF,)),
        ],
    )
    def body(text_hbm, mask_hbm, table_hbm, out_hbm,
             text_v, mask_v, idx_v, rows_v, gsem, wsem):
        nc = jax.lax.axis_size("c")
        wid = lax.axis_index("s") * nc + lax.axis_index("c")
        base = wid * bpw
        pltpu.sync_copy(text_hbm.at[pl.ds(base, bpw)], text_v)
        pltpu.sync_copy(mask_hbm.at[pl.ds(base, bpw)], mask_v)

        def compute_chunk(j, _):
            for k in range(CHUNK // LANES):
                off = j * CHUNK + k * LANES
                t = text_v[pl.ds(off, LANES)]
                m = mask_v[pl.ds(off, LANES)]
                idx_v[j, pl.ds(k * LANES, LANES)] = t * m
            return 0

        lax.fori_loop(0, nchunks, compute_chunk, 0)

        def gstart(b, j):
            pltpu.make_async_copy(
                table_hbm.at[idx_v.at[j]], rows_v.at[b], gsem.at[b]).start()

        def gwait(b, j):
            pltpu.make_async_copy(
                table_hbm.at[idx_v.at[j]], rows_v.at[b], gsem.at[b]).wait()

        def wstart(b, j):
            pltpu.make_async_copy(
                rows_v.at[b], out_hbm.at[pl.ds(base + j * CHUNK, CHUNK)],
                wsem.at[b]).start()

        def wwait(b, j):
            pltpu.make_async_copy(
                rows_v.at[b], out_hbm.at[pl.ds(base + j * CHUNK, CHUNK)],
                wsem.at[b]).wait()

        for b in range(NBUF):
            gstart(b, b)

        def pipeline_round(r, _):
            j0 = r * NBUF
            for b in range(NBUF):
                gwait(b, j0 + b)
                wstart(b, j0 + b)
            jn0 = j0 + NBUF
            for b in range(NBUF):

                @pl.when(jn0 + b < nchunks)
                def _():
                    wwait(b, j0 + b)
                    gstart(b, jn0 + b)

            return 0

        lax.fori_loop(0, nrounds, pipeline_round, 0)
        for b in range(NBUF):
            wwait(b, nchunks - NBUF + b)

    return body


def kernel(text, word_mask, embedding_matrix):
    B, L = text.shape
    n = B * L
    n_workers = 32
    bpw = n // n_workers
    text_flat = text.reshape(n).astype(jnp.int32)
    mask_flat = word_mask.reshape(n).astype(jnp.int32)
    out = _sc_gather(text_flat, mask_flat, embedding_matrix,
                     n_workers=n_workers, bpw=bpw)(
        text_flat, mask_flat, embedding_matrix)
    return out.reshape(B, L, DIM)


# speedup vs baseline: 5.4286x; 3.4361x over previous
import functools

import jax
import jax.numpy as jnp
from jax import lax
from jax.experimental import pallas as pl
from jax.experimental.pallas import tpu as pltpu
from jax.experimental.pallas import tpu_sc as plsc

DIM = 64
LANES = 16

_GATHER_DNUMS = lax.GatherDimensionNumbers(
    offset_dims=(), collapsed_slice_dims=(0,), start_index_map=(0,))


def _bcast_lane(x16, r):
    idx = jnp.full((LANES, 1), r, jnp.int32)
    return lax.gather(x16, idx, _GATHER_DNUMS, (1,),
                      mode=lax.GatherScatterMode.PROMISE_IN_BOUNDS)
CHUNK = 128
NBUF = 5
N_WORKERS = 32


def _sc_gather(n):
    bpw = n // N_WORKERS
    nchunks = bpw // CHUNK
    nrounds = nchunks // NBUF
    mesh = plsc.VectorSubcoreMesh(core_axis_name="c", subcore_axis_name="s")

    @functools.partial(
        pl.kernel,
        mesh=mesh,
        compiler_params=pltpu.CompilerParams(use_tc_tiling_on_sc=False),
        out_type=jax.ShapeDtypeStruct((n, DIM), jnp.float32),
        scratch_types=[
            pltpu.VMEM((nchunks, CHUNK), jnp.int32),
            pltpu.VMEM((nchunks, CHUNK), jnp.int32),
            pltpu.VMEM((NBUF, CHUNK, DIM), jnp.float32),
            pltpu.SemaphoreType.DMA((NBUF,)),
            pltpu.SemaphoreType.DMA((NBUF,)),
        ],
    )
    def body(text_hbm, mask_hbm, table_hbm, out_hbm, idx_v, mask_v, rows_v, gsem, wsem):
        nc = jax.lax.axis_size("c")
        wid = lax.axis_index("s") * nc + lax.axis_index("c")
        base = wid * nchunks
        pltpu.sync_copy(text_hbm.at[pl.ds(base, nchunks)], idx_v)
        pltpu.sync_copy(mask_hbm.at[pl.ds(base, nchunks)], mask_v)
        rbase = wid * bpw

        def gstart(b, j):
            pltpu.make_async_copy(
                table_hbm.at[idx_v.at[j]], rows_v.at[b], gsem.at[b]).start()

        def gwait(b, j):
            pltpu.make_async_copy(
                table_hbm.at[idx_v.at[j]], rows_v.at[b], gsem.at[b]).wait()

        def wstart(b, j):
            pltpu.make_async_copy(
                rows_v.at[b], out_hbm.at[pl.ds(rbase + j * CHUNK, CHUNK)],
                wsem.at[b]).start()

        def wwait(b, j):
            pltpu.make_async_copy(
                rows_v.at[b], out_hbm.at[pl.ds(rbase + j * CHUNK, CHUNK)],
                wsem.at[b]).wait()

        def mask_rows(b, j):
            def group(g, _):
                m16 = mask_v[j, pl.ds(g * LANES, LANES)].astype(jnp.float32)
                for r in range(LANES):
                    mg = _bcast_lane(m16, r)
                    row = g * LANES + r
                    for k in range(DIM // LANES):
                        sl = pl.ds(k * LANES, LANES)
                        rows_v[b, row, sl] = rows_v[b, row, sl] * mg
                return 0

            lax.fori_loop(0, CHUNK // LANES, group, 0)

        for b in range(NBUF):
            gstart(b, b)

        def pipeline_round(r, _):
            j0 = r * NBUF
            for b in range(NBUF):
                gwait(b, j0 + b)
                mask_rows(b, j0 + b)
                wstart(b, j0 + b)
            jn0 = j0 + NBUF
            for b in range(NBUF):

                @pl.when(jn0 + b < nchunks)
                def _():
                    wwait(b, j0 + b)
                    gstart(b, jn0 + b)

            return 0

        lax.fori_loop(0, nrounds, pipeline_round, 0)
        for b in range(NBUF):
            wwait(b, nchunks - NBUF + b)

    return body


def kernel(text, word_mask, embedding_matrix):
    B, L = text.shape
    n = B * L
    text2 = text.reshape(n // CHUNK, CHUNK).astype(jnp.int32)
    mask2 = word_mask.reshape(n // CHUNK, CHUNK).astype(jnp.int32)
    out = _sc_gather(n)(text2, mask2, embedding_matrix)
    return out.reshape(B, L, DIM)
